# baseline (speedup 1.0000x reference)
import functools

import jax
import jax.numpy as jnp
from jax import lax
from jax.experimental import pallas as pl
from jax.experimental.pallas import tpu as pltpu
from jax.experimental.pallas import tpu_sc as plsc

BATCH = 16384
N_FIELDS = 26
FIELD_VOCAB = 100000
EMB = 16
IN_DIM = (N_FIELDS + 2) * EMB

NC = 2
NS = 16
NW = NC * NS
ROWS_PER_W = BATCH // NW
CHUNK = 128
NCHUNK = ROWS_PER_W // CHUNK


def _gather_body(user_ids, item_ids, sf_flat, user_table, item_table,
                 sparse_flat, out, idx_v, rows_v, sem):
    wid = lax.axis_index("s") * NC + lax.axis_index("c")
    base = wid * ROWS_PER_W

    def chunk_step(c, carry):
        rb = base + c * CHUNK

        pltpu.sync_copy(user_ids.at[pl.ds(rb, CHUNK)], idx_v)
        pltpu.async_copy(user_table.at[idx_v], rows_v, sem).wait()
        pltpu.sync_copy(rows_v, out.at[0, pl.ds(rb, CHUNK), :])

        pltpu.sync_copy(item_ids.at[pl.ds(rb, CHUNK)], idx_v)
        pltpu.async_copy(item_table.at[idx_v], rows_v, sem).wait()
        pltpu.sync_copy(rows_v, out.at[1, pl.ds(rb, CHUNK), :])

        def field_step(f, carry2):
            pltpu.sync_copy(sf_flat.at[f, pl.ds(rb, CHUNK)], idx_v)
            pltpu.async_copy(sparse_flat.at[idx_v], rows_v, sem).wait()
            pltpu.sync_copy(rows_v, out.at[f + 2, pl.ds(rb, CHUNK), :])
            return carry2

        lax.fori_loop(0, N_FIELDS, field_step, 0)
        return carry

    lax.fori_loop(0, NCHUNK, chunk_step, 0)


_sc_gather = functools.partial(
    pl.kernel,
    mesh=plsc.VectorSubcoreMesh(core_axis_name="c", subcore_axis_name="s"),
    compiler_params=pltpu.CompilerParams(use_tc_tiling_on_sc=False),
    out_type=jax.ShapeDtypeStruct((N_FIELDS + 2, BATCH, EMB), jnp.float32),
    scratch_types=[
        pltpu.VMEM((CHUNK,), jnp.int32),
        pltpu.VMEM((CHUNK, EMB), jnp.float32),
        pltpu.SemaphoreType.DMA,
    ],
)(_gather_body)


def kernel(user_ids, item_ids, sparse_features, user_table, item_table,
           sparse_tables, W1, b1, W2, b2, W3, b3, W4, b4):
    sf_flat = (sparse_features.astype(jnp.int32)
               + jnp.arange(N_FIELDS, dtype=jnp.int32)[None, :] * FIELD_VOCAB).T
    sparse_flat = sparse_tables.reshape(N_FIELDS * FIELD_VOCAB, EMB)
    g = _sc_gather(user_ids, item_ids, sf_flat, user_table, item_table,
                   sparse_flat)
    x = jnp.swapaxes(g, 0, 1).reshape(BATCH, IN_DIM)
    h = jnp.maximum(x @ W1 + b1, 0.0)
    h = jnp.maximum(h @ W2 + b2, 0.0)
    h = jnp.maximum(h @ W3 + b3, 0.0)
    deep = (h @ W4 + b4)[:, 0]
    s = jnp.sum(x, axis=1)
    sq = jnp.sum(x * x, axis=1)
    fm = 0.5 * (s * s - sq)
    return jax.nn.sigmoid(deep + fm)
